# speedup vs baseline: 1.1406x; 1.1406x over previous
import functools

import jax
import jax.numpy as jnp
from jax.experimental import pallas as pl
from jax.experimental.pallas import tpu as pltpu

DIM = 4096
NUM_EXPERTS = 64
TOP_K = 8
TOKEN_BLOCK = 512


def _gate_body(x_ref, wt_ref, w_out_ref, i_out_ref):
    logits = jax.lax.dot_general(
        x_ref[...], wt_ref[...],
        dimension_numbers=(((1,), (0,)), ((), ())),
        preferred_element_type=jnp.float32,
    )
    tb = logits.shape[0]
    col = jax.lax.broadcasted_iota(jnp.int32, logits.shape, 1)
    col8 = jax.lax.broadcasted_iota(jnp.int32, (tb, TOP_K), 1)
    neg_inf = jnp.float32(float("-inf"))

    work = logits
    top_v = jnp.zeros((tb, TOP_K), jnp.float32)
    top_i = jnp.zeros((tb, TOP_K), jnp.int32)
    for k in range(TOP_K):
        m = jnp.max(work, axis=1, keepdims=True)
        idx = jnp.min(jnp.where(work == m, col, NUM_EXPERTS), axis=1,
                      keepdims=True)
        top_v = jnp.where(col8 == k, m, top_v)
        top_i = jnp.where(col8 == k, idx, top_i)
        work = jnp.where(col == idx, neg_inf, work)

    m0 = jnp.max(top_v, axis=1, keepdims=True)
    e = jnp.exp(top_v - m0)
    w_out_ref[...] = e / jnp.sum(e, axis=1, keepdims=True)
    i_out_ref[...] = top_i


@functools.partial(jax.jit, static_argnames=("interpret",))
def kernel(x, W, interpret=False):
    b, n, d = x.shape
    tokens = b * n
    xt = x.reshape(tokens, d)
    wt = W.T
    grid = (tokens // TOKEN_BLOCK,)
    weights, indices = pl.pallas_call(
        _gate_body,
        grid=grid,
        in_specs=[
            pl.BlockSpec((TOKEN_BLOCK, d), lambda i: (i, 0)),
            pl.BlockSpec((d, NUM_EXPERTS), lambda i: (0, 0)),
        ],
        out_specs=[
            pl.BlockSpec((TOKEN_BLOCK, TOP_K), lambda i: (i, 0)),
            pl.BlockSpec((TOKEN_BLOCK, TOP_K), lambda i: (i, 0)),
        ],
        out_shape=[
            jax.ShapeDtypeStruct((tokens, TOP_K), jnp.float32),
            jax.ShapeDtypeStruct((tokens, TOP_K), jnp.int32),
        ],
        compiler_params=pltpu.CompilerParams(
            dimension_semantics=("arbitrary",),
        ),
        interpret=interpret,
    )(xt, wt)
    return weights.reshape(b, n, TOP_K), indices.reshape(b, n, TOP_K)


# speedup vs baseline: 1.6763x; 1.4697x over previous
import functools

import jax
import jax.numpy as jnp
from jax.experimental import pallas as pl
from jax.experimental.pallas import tpu as pltpu

DIM = 4096
NUM_EXPERTS = 64
TOP_K = 8
TOKEN_BLOCK = 512


def _gate_body(x_ref, wt_ref, w_out_ref, i_out_ref):
    logits = jax.lax.dot_general(
        x_ref[...], wt_ref[...],
        dimension_numbers=(((1,), (0,)), ((), ())),
        preferred_element_type=jnp.float32,
    )
    tb = logits.shape[0]
    lt = logits.T
    row = jax.lax.broadcasted_iota(jnp.int32, lt.shape, 0)
    row8 = jax.lax.broadcasted_iota(jnp.int32, (TOP_K, tb), 0)
    neg_inf = jnp.float32(float("-inf"))

    work = lt
    top_v = jnp.zeros((TOP_K, tb), jnp.float32)
    top_i = jnp.zeros((TOP_K, tb), jnp.int32)
    for k in range(TOP_K):
        m = jnp.max(work, axis=0, keepdims=True)
        idx = jnp.min(jnp.where(work == m, row, NUM_EXPERTS), axis=0,
                      keepdims=True)
        top_v = jnp.where(row8 == k, m, top_v)
        top_i = jnp.where(row8 == k, idx, top_i)
        work = jnp.where(row == idx, neg_inf, work)

    m0 = jnp.max(top_v, axis=0, keepdims=True)
    e = jnp.exp(top_v - m0)
    w_out_ref[...] = e / jnp.sum(e, axis=0, keepdims=True)
    i_out_ref[...] = top_i


@functools.partial(jax.jit, static_argnames=("interpret",))
def kernel(x, W, interpret=False):
    b, n, d = x.shape
    tokens = b * n
    xt = x.reshape(tokens, d)
    wt = W.T
    grid = (tokens // TOKEN_BLOCK,)
    weights_t, indices_t = pl.pallas_call(
        _gate_body,
        grid=grid,
        in_specs=[
            pl.BlockSpec((TOKEN_BLOCK, d), lambda i: (i, 0)),
            pl.BlockSpec((d, NUM_EXPERTS), lambda i: (0, 0)),
        ],
        out_specs=[
            pl.BlockSpec((TOP_K, TOKEN_BLOCK), lambda i: (0, i)),
            pl.BlockSpec((TOP_K, TOKEN_BLOCK), lambda i: (0, i)),
        ],
        out_shape=[
            jax.ShapeDtypeStruct((TOP_K, tokens), jnp.float32),
            jax.ShapeDtypeStruct((TOP_K, tokens), jnp.int32),
        ],
        compiler_params=pltpu.CompilerParams(
            dimension_semantics=("arbitrary",),
        ),
        interpret=interpret,
    )(xt, wt)
    weights = weights_t.T.reshape(b, n, TOP_K)
    indices = indices_t.T.reshape(b, n, TOP_K)
    return weights, indices


# speedup vs baseline: 1.7397x; 1.0378x over previous
import functools

import jax
import jax.numpy as jnp
from jax.experimental import pallas as pl
from jax.experimental.pallas import tpu as pltpu

DIM = 4096
NUM_EXPERTS = 64
TOP_K = 8
TOKEN_BLOCK = 1024


def _gate_body(x_ref, wt_ref, w_out_ref, i_out_ref):
    logits = jax.lax.dot_general(
        x_ref[...], wt_ref[...],
        dimension_numbers=(((1,), (0,)), ((), ())),
        preferred_element_type=jnp.float32,
    )
    tb = logits.shape[0]
    lt = logits.T
    row = jax.lax.broadcasted_iota(jnp.int32, lt.shape, 0)
    row8 = jax.lax.broadcasted_iota(jnp.int32, (TOP_K, tb), 0)
    neg_inf = jnp.float32(float("-inf"))

    work = lt
    top_v = jnp.zeros((TOP_K, tb), jnp.float32)
    top_i = jnp.zeros((TOP_K, tb), jnp.int32)
    for k in range(TOP_K):
        m = jnp.max(work, axis=0, keepdims=True)
        idx = jnp.min(jnp.where(work == m, row, NUM_EXPERTS), axis=0,
                      keepdims=True)
        top_v = jnp.where(row8 == k, m, top_v)
        top_i = jnp.where(row8 == k, idx, top_i)
        work = jnp.where(row == idx, neg_inf, work)

    m0 = jnp.max(top_v, axis=0, keepdims=True)
    e = jnp.exp(top_v - m0)
    w_out_ref[...] = e / jnp.sum(e, axis=0, keepdims=True)
    i_out_ref[...] = top_i


@functools.partial(jax.jit, static_argnames=("interpret",))
def kernel(x, W, interpret=False):
    b, n, d = x.shape
    tokens = b * n
    xt = x.reshape(tokens, d)
    wt = W.T
    grid = (tokens // TOKEN_BLOCK,)
    weights_t, indices_t = pl.pallas_call(
        _gate_body,
        grid=grid,
        in_specs=[
            pl.BlockSpec((TOKEN_BLOCK, d), lambda i: (i, 0)),
            pl.BlockSpec((d, NUM_EXPERTS), lambda i: (0, 0)),
        ],
        out_specs=[
            pl.BlockSpec((TOP_K, TOKEN_BLOCK), lambda i: (0, i)),
            pl.BlockSpec((TOP_K, TOKEN_BLOCK), lambda i: (0, i)),
        ],
        out_shape=[
            jax.ShapeDtypeStruct((TOP_K, tokens), jnp.float32),
            jax.ShapeDtypeStruct((TOP_K, tokens), jnp.int32),
        ],
        compiler_params=pltpu.CompilerParams(
            dimension_semantics=("arbitrary",),
        ),
        interpret=interpret,
    )(xt, wt)
    weights = weights_t.T.reshape(b, n, TOP_K)
    indices = indices_t.T.reshape(b, n, TOP_K)
    return weights, indices
